# Initial kernel scaffold; baseline (speedup 1.0000x reference)
import jax
import jax.numpy as jnp
from jax.experimental import pallas as pl


def kernel(output, action_emb):
    raise NotImplementedError("write your pallas kernel here")


# speedup vs baseline: 2.4441x; 2.4441x over previous
import functools

import jax
import jax.numpy as jnp
from jax import lax
from jax.experimental import pallas as pl
from jax.experimental.pallas import tpu as pltpu
from jax.experimental.pallas import tpu_sc as plsc

_NC = 2
_NS = 16
_NW = _NC * _NS
_L = 16

_B = 4096 * 26 * 20
_D = 64
_BPW = _B // _NW
_C = 512
_NCH = _BPW // _C
_KI = _C // 128

_ZROW = 42
_VPAD = 48


@functools.partial(
    pl.kernel,
    out_type=(
        jax.ShapeDtypeStruct((_B, _D), jnp.float32),
        jax.ShapeDtypeStruct((_B,), jnp.int32),
    ),
    mesh=plsc.VectorSubcoreMesh(core_axis_name="c", subcore_axis_name="s"),
    compiler_params=pltpu.CompilerParams(use_tc_tiling_on_sc=False),
    scratch_types=[
        pltpu.VMEM((_C,), jnp.int32),
        pltpu.VMEM((_KI, 128), jnp.int32),
        pltpu.VMEM((_C,), jnp.int32),
        pltpu.VMEM((_C, _D), jnp.float32),
        pltpu.SemaphoreType.DMA,
    ],
)
def _emb_lookup(table_hbm, labels_hbm, out_hbm, mask_hbm,
                raw_v, idx_v, mask_v, rows_v, sem):
    wid = lax.axis_index("s") * _NC + lax.axis_index("c")
    base0 = wid * _BPW

    def chunk_body(i, carry):
        base = base0 + i * _C
        pltpu.sync_copy(labels_hbm.at[pl.ds(base, _C)], raw_v)

        def vec_body(j, carry2):
            lab = raw_v[pl.ds(j * _L, _L)]
            is_pad = lab == jnp.full((_L,), -1, jnp.int32)
            mask_v[pl.ds(j * _L, _L)] = jnp.where(
                is_pad, jnp.zeros((_L,), jnp.int32), jnp.ones((_L,), jnp.int32))
            fixed = jnp.where(is_pad, jnp.full((_L,), _ZROW, jnp.int32), lab)
            idx_v[(j * _L) // 128, pl.ds((j * _L) % 128, _L)] = fixed
            return carry2

        lax.fori_loop(0, _C // _L, vec_body, 0)

        gathers = [
            pltpu.async_copy(table_hbm.at[idx_v.at[k]],
                             rows_v.at[pl.ds(k * 128, 128)], sem)
            for k in range(_KI)
        ]
        pltpu.sync_copy(mask_v, mask_hbm.at[pl.ds(base, _C)])
        for g in gathers:
            g.wait()
        pltpu.sync_copy(rows_v, out_hbm.at[pl.ds(base, _C)])
        return carry

    lax.fori_loop(0, _NCH, chunk_body, 0)


def kernel(output, action_emb):
    labels = output[0].reshape(_B)
    table = jnp.concatenate(
        [action_emb, jnp.zeros((_VPAD - action_emb.shape[0], _D), jnp.float32)])
    emb_flat, mask_flat = _emb_lookup(table, labels)
    return (emb_flat.reshape(4096, 26, 20, _D), mask_flat.reshape(4096, 26, 20))


# speedup vs baseline: 5.8366x; 2.3880x over previous
import functools

import jax
import jax.numpy as jnp
from jax import lax
from jax.experimental import pallas as pl
from jax.experimental.pallas import tpu as pltpu
from jax.experimental.pallas import tpu_sc as plsc

_NC = 2
_NS = 16
_NW = _NC * _NS
_L = 16

_B = 4096 * 26 * 20
_D = 64
_BPW = _B // _NW
_C = 512
_NCH = _BPW // _C
_KI = _C // 128

_ZROW = 42
_VPAD = 48


@functools.partial(
    pl.kernel,
    out_type=(
        jax.ShapeDtypeStruct((_B, _D), jnp.float32),
        jax.ShapeDtypeStruct((_B,), jnp.int32),
    ),
    mesh=plsc.VectorSubcoreMesh(core_axis_name="c", subcore_axis_name="s"),
    compiler_params=pltpu.CompilerParams(use_tc_tiling_on_sc=False),
    scratch_types=[
        pltpu.VMEM_SHARED((_VPAD, _D), jnp.float32),
        pltpu.VMEM((_VPAD, _D), jnp.float32),
        pltpu.VMEM((_C,), jnp.int32),
        pltpu.VMEM((2, _KI, 128), jnp.int32),
        pltpu.VMEM((2, _C), jnp.int32),
        pltpu.VMEM((2, _C, _D), jnp.float32),
        pltpu.SemaphoreType.DMA,
        pltpu.SemaphoreType.DMA,
        pltpu.SemaphoreType.DMA,
        pltpu.SemaphoreType.DMA,
        pltpu.SemaphoreType.DMA,
        pltpu.SemaphoreType.DMA,
    ],
)
def _emb_lookup(table_hbm, labels_hbm, out_hbm, mask_hbm,
                table_sp, table_v, raw_v, idx_v, mask_v, rows_v,
                gsem0, gsem1, osem0, osem1, msem0, msem1):
    cid = lax.axis_index("c")
    sid = lax.axis_index("s")
    wid = sid * _NC + cid
    base0 = wid * _BPW

    @pl.when(sid == 0)
    def _():
        pltpu.sync_copy(table_hbm, table_v)
        pltpu.sync_copy(table_v, table_sp)

    plsc.subcore_barrier()

    gsems = (gsem0, gsem1)
    osems = (osem0, osem1)
    msems = (msem0, msem1)

    def load_and_preprocess(i, b):
        pltpu.sync_copy(labels_hbm.at[pl.ds(base0 + i * _C, _C)], raw_v)

        def vec_body(j, carry):
            lab = raw_v[pl.ds(j * _L, _L)]
            is_pad = lab == jnp.full((_L,), -1, jnp.int32)
            mask_v[b, pl.ds(j * _L, _L)] = jnp.where(
                is_pad, jnp.zeros((_L,), jnp.int32), jnp.ones((_L,), jnp.int32))
            fixed = jnp.where(is_pad, jnp.full((_L,), _ZROW, jnp.int32), lab)
            idx_v[b, (j * _L) // 128, pl.ds((j * _L) % 128, _L)] = fixed
            return carry

        lax.fori_loop(0, _C // _L, vec_body, 0)

    def fire_gather(b):
        for k in range(_KI):
            pltpu.async_copy(table_sp.at[idx_v.at[b].at[k]],
                             rows_v.at[b].at[pl.ds(k * 128, 128)], gsems[b])

    def wait_gather(b):
        for k in range(_KI):
            pltpu.make_async_copy(table_sp.at[idx_v.at[b].at[k]],
                                  rows_v.at[b].at[pl.ds(k * 128, 128)],
                                  gsems[b]).wait()

    def fire_out(i, b):
        pltpu.async_copy(rows_v.at[b], out_hbm.at[pl.ds(base0 + i * _C, _C)],
                         osems[b])
        pltpu.async_copy(mask_v.at[b], mask_hbm.at[pl.ds(base0 + i * _C, _C)],
                         msems[b])

    def wait_out(i, b):
        pltpu.make_async_copy(rows_v.at[b],
                              out_hbm.at[pl.ds(base0 + i * _C, _C)],
                              osems[b]).wait()
        pltpu.make_async_copy(mask_v.at[b],
                              mask_hbm.at[pl.ds(base0 + i * _C, _C)],
                              msems[b]).wait()

    def step(i, b):
        b2 = 1 - b
        wait_gather(b2)
        fire_out(i - 1, b2)

        @pl.when(i >= 2)
        def _():
            wait_out(i - 2, b)

        load_and_preprocess(i, b)
        fire_gather(b)

    load_and_preprocess(0, 0)
    fire_gather(0)

    def pair_body(g, carry):
        step(2 * g + 1, 1)

        @pl.when(2 * g + 2 < _NCH)
        def _():
            step(2 * g + 2, 0)

        return carry

    lax.fori_loop(0, (_NCH + 1) // 2, pair_body, 0)
    wait_gather(1)
    fire_out(_NCH - 1, 1)
    wait_out(_NCH - 2, 0)
    wait_out(_NCH - 1, 1)


def kernel(output, action_emb):
    labels = output[0].reshape(_B)
    table = jnp.concatenate(
        [action_emb, jnp.zeros((_VPAD - action_emb.shape[0], _D), jnp.float32)])
    emb_flat, mask_flat = _emb_lookup(table, labels)
    return (emb_flat.reshape(4096, 26, 20, _D), mask_flat.reshape(4096, 26, 20))
